# Initial kernel scaffold; baseline (speedup 1.0000x reference)
import jax
import jax.numpy as jnp
from jax.experimental import pallas as pl


def kernel(table, hash_indices):
    raise NotImplementedError("write your pallas kernel here")


# speedup vs baseline: 22.1809x; 22.1809x over previous
import functools

import jax
import jax.numpy as jnp
import numpy as np
from jax import lax
from jax.experimental import pallas as pl
from jax.experimental.pallas import tpu as pltpu
from jax.experimental.pallas import tpu_sc as plsc

_PRIMES = [99991, 100003, 100019, 100043, 100057, 100069, 100103, 100109]
_EMBED_DIM = 16

_NC = 2
_NS = 16
_NW = _NC * _NS
_LANES = 16

_CHUNK = 1024
_GROW = 128
_NG = _CHUNK // _GROW


def _offsets_np():
    offs = [0]
    for p in _PRIMES[:-1]:
        offs.append(offs[-1] + p)
    return np.asarray(offs, dtype=np.int32)


@functools.partial(jax.jit, static_argnames=("n_chunks",))
def _sc_gather(table, idx2, off16, n_chunks):
    b_total = idx2.shape[0] * idx2.shape[1]
    mesh = plsc.VectorSubcoreMesh(core_axis_name="c", subcore_axis_name="s")

    @functools.partial(
        pl.kernel,
        mesh=mesh,
        out_type=jax.ShapeDtypeStruct((b_total, _EMBED_DIM), jnp.float32),
        compiler_params=pltpu.CompilerParams(use_tc_tiling_on_sc=False),
        scratch_types=[
            pltpu.VMEM((_NG, _GROW), jnp.int32),
            pltpu.VMEM((_CHUNK, _EMBED_DIM), jnp.float32),
            pltpu.VMEM((_LANES,), jnp.int32),
            pltpu.SemaphoreType.DMA,
        ],
    )
    def body(table_hbm, idx_hbm, off_hbm, out_hbm, idx_v, rows_v, off_v, sem):
        wid = lax.axis_index("s") * _NC + lax.axis_index("c")
        pltpu.sync_copy(off_hbm, off_v)
        off_reg = off_v[...]

        def chunk_body(c, _):
            chunk = wid * n_chunks + c
            row0 = chunk * (_CHUNK // _GROW)
            pltpu.sync_copy(idx_hbm.at[pl.ds(row0, _NG)], idx_v)
            for j in range(_NG):
                for k in range(_GROW // _LANES):
                    sl = pl.ds(k * _LANES, _LANES)
                    idx_v[j, sl] = idx_v[j, sl] + off_reg
            handles = []
            for j in range(_NG):
                handles.append(
                    pltpu.async_copy(
                        table_hbm.at[idx_v.at[j]],
                        rows_v.at[pl.ds(j * _GROW, _GROW)],
                        sem,
                    )
                )
            for h in handles:
                h.wait()
            pltpu.sync_copy(rows_v, out_hbm.at[pl.ds(chunk * _CHUNK, _CHUNK)])
            return _

        lax.fori_loop(0, n_chunks, chunk_body, 0)

    return body(table, idx2, off16)


def kernel(table, hash_indices):
    shape = hash_indices.shape
    b_total = int(np.prod(shape))
    idx2 = hash_indices.reshape(b_total // _GROW, _GROW).astype(jnp.int32)
    off16 = jnp.asarray(np.tile(_offsets_np(), 2), dtype=jnp.int32)
    n_chunks = b_total // (_NW * _CHUNK)
    out = _sc_gather(table, idx2, off16, n_chunks)
    return out.reshape(*shape, _EMBED_DIM)


# speedup vs baseline: 23.3142x; 1.0511x over previous
import functools

import jax
import jax.numpy as jnp
import numpy as np
from jax import lax
from jax.experimental import pallas as pl
from jax.experimental.pallas import tpu as pltpu
from jax.experimental.pallas import tpu_sc as plsc

_PRIMES = [99991, 100003, 100019, 100043, 100057, 100069, 100103, 100109]
_EMBED_DIM = 16

_NC = 2
_NS = 16
_NW = _NC * _NS
_LANES = 16

_CHUNK = 1024
_GROW = 128
_NG = _CHUNK // _GROW
_NBUF = 2


def _offsets_np():
    offs = [0]
    for p in _PRIMES[:-1]:
        offs.append(offs[-1] + p)
    return np.asarray(offs, dtype=np.int32)


@functools.partial(jax.jit, static_argnames=("n_chunks",))
def _sc_gather(table, idx2, off16, n_chunks):
    b_total = idx2.shape[0] * idx2.shape[1]
    last_chunk = b_total // _CHUNK - 1
    mesh = plsc.VectorSubcoreMesh(core_axis_name="c", subcore_axis_name="s")

    @functools.partial(
        pl.kernel,
        mesh=mesh,
        out_type=jax.ShapeDtypeStruct((b_total, _EMBED_DIM), jnp.float32),
        compiler_params=pltpu.CompilerParams(use_tc_tiling_on_sc=False),
        scratch_types=[
            pltpu.VMEM((_NBUF, _NG, _GROW), jnp.int32),
            pltpu.VMEM((_NBUF, _CHUNK, _EMBED_DIM), jnp.float32),
            pltpu.VMEM((_LANES,), jnp.int32),
            pltpu.SemaphoreType.DMA,
            pltpu.SemaphoreType.DMA,
            pltpu.SemaphoreType.DMA,
            pltpu.SemaphoreType.DMA,
            pltpu.SemaphoreType.DMA,
            pltpu.SemaphoreType.DMA,
        ],
    )
    def body(table_hbm, idx_hbm, off_hbm, out_hbm, idx_v, rows_v, off_v,
             si0, si1, sg0, sg1, so0, so1):
        sem_i = (si0, si1)
        sem_g = (sg0, sg1)
        sem_o = (so0, so1)
        wid = lax.axis_index("s") * _NC + lax.axis_index("c")
        base_chunk = wid * n_chunks
        pltpu.sync_copy(off_hbm, off_v)

        def fire_idx(slot, chunk):
            pltpu.async_copy(idx_hbm.at[pl.ds(chunk * _NG, _NG)],
                             idx_v.at[slot], sem_i[slot])

        def wait_idx(slot):
            pltpu.make_async_copy(idx_hbm.at[pl.ds(0, _NG)],
                                  idx_v.at[slot], sem_i[slot]).wait()

        def do_adds(slot):
            off_reg = off_v[...]
            for j in range(_NG):
                for k in range(_GROW // _LANES):
                    sl = pl.ds(k * _LANES, _LANES)
                    idx_v[slot, j, sl] = idx_v[slot, j, sl] + off_reg

        def fire_gathers(slot):
            for j in range(_NG):
                pltpu.async_copy(
                    table_hbm.at[idx_v.at[slot, j]],
                    rows_v.at[slot, pl.ds(j * _GROW, _GROW)],
                    sem_g[slot],
                )

        def wait_gathers(slot):
            pltpu.make_async_copy(table_hbm.at[pl.ds(0, _CHUNK)],
                                  rows_v.at[slot], sem_g[slot]).wait()

        def fire_out(slot, chunk):
            pltpu.async_copy(rows_v.at[slot],
                             out_hbm.at[pl.ds(chunk * _CHUNK, _CHUNK)],
                             sem_o[slot])

        def wait_out(slot):
            pltpu.make_async_copy(rows_v.at[slot],
                                  out_hbm.at[pl.ds(0, _CHUNK)],
                                  sem_o[slot]).wait()

        for b in range(_NBUF):
            fire_idx(b, base_chunk + b)
        for b in range(_NBUF):
            wait_idx(b)
            do_adds(b)
            fire_gathers(b)
        for b in range(_NBUF):
            wait_gathers(b)
            fire_out(b, base_chunk + b)
            fire_idx(b, jnp.minimum(base_chunk + _NBUF + b, last_chunk))

        def loop_body(g, _):
            c0 = base_chunk + _NBUF * g
            for b in range(_NBUF):
                wait_idx(b)
                do_adds(b)
                wait_out(b)
                fire_gathers(b)
            for b in range(_NBUF):
                wait_gathers(b)
                fire_out(b, c0 + b)
                fire_idx(b, jnp.minimum(c0 + _NBUF + b, last_chunk))
            return _

        lax.fori_loop(1, n_chunks // _NBUF, loop_body, 0)

        for b in range(_NBUF):
            wait_idx(b)
            wait_out(b)

    return body(table, idx2, off16)


def kernel(table, hash_indices):
    shape = hash_indices.shape
    b_total = int(np.prod(shape))
    idx2 = hash_indices.reshape(b_total // _GROW, _GROW).astype(jnp.int32)
    off16 = jnp.asarray(np.tile(_offsets_np(), 2), dtype=jnp.int32)
    n_chunks = b_total // (_NW * _CHUNK)
    out = _sc_gather(table, idx2, off16, n_chunks)
    return out.reshape(*shape, _EMBED_DIM)


# speedup vs baseline: 40.6119x; 1.7419x over previous
import functools

import jax
import jax.numpy as jnp
import numpy as np
from jax import lax
from jax.experimental import pallas as pl
from jax.experimental.pallas import tpu as pltpu
from jax.experimental.pallas import tpu_sc as plsc

_PRIMES = [99991, 100003, 100019, 100043, 100057, 100069, 100103, 100109]
_EMBED_DIM = 16

_NC = 2
_NS = 16
_NW = _NC * _NS
_LANES = 16

_BQ = 1024
_GROW = 128
_NG = _BQ // _GROW
_NBUF = 2


def _offsets_np():
    offs = [0]
    for p in _PRIMES[:-1]:
        offs.append(offs[-1] + p)
    return np.asarray(offs, dtype=np.int32)


@functools.partial(jax.jit, static_argnames=("units_per_w",))
def _sc_gather(table, idx6, off16, units_per_w):
    s_dim, c_dim, h_dim, l_dim = idx6.shape
    n_units = s_dim * h_dim * (c_dim * l_dim // _BQ)
    mesh = plsc.VectorSubcoreMesh(core_axis_name="c", subcore_axis_name="s")

    @functools.partial(
        pl.kernel,
        mesh=mesh,
        out_type=jax.ShapeDtypeStruct(
            (s_dim, h_dim, 2, c_dim, 8, l_dim), jnp.float32),
        compiler_params=pltpu.CompilerParams(
            use_tc_tiling_on_sc=False, needs_layout_passes=False),
        scratch_types=[
            pltpu.VMEM((_NBUF, _NG, _GROW), jnp.int32),
            pltpu.VMEM((_NBUF, _BQ, _EMBED_DIM), jnp.float32),
            pltpu.VMEM((_NBUF, 2, _NG, 8, _GROW), jnp.float32),
            pltpu.VMEM((8, _LANES), jnp.int32),
            pltpu.SemaphoreType.DMA,
            pltpu.SemaphoreType.DMA,
            pltpu.SemaphoreType.DMA,
            pltpu.SemaphoreType.DMA,
            pltpu.SemaphoreType.DMA,
            pltpu.SemaphoreType.DMA,
        ],
    )
    def body(table_hbm, idx_hbm, off_hbm, out_hbm, idx_v, rows_v, trans_v,
             off_v, si0, si1, sg0, sg1, so0, so1):
        sem_i = (si0, si1)
        sem_g = (sg0, sg1)
        sem_o = (so0, so1)
        wid = lax.axis_index("s") * _NC + lax.axis_index("c")
        base_u = wid * units_per_w
        pltpu.sync_copy(off_hbm, off_v)
        iota16 = lax.iota(jnp.int32, _LANES)

        def decode(u):
            pair = u >> 2
            q = u & 3
            return pair >> 3, pair & 7, q * _NG

        def fire_idx(slot, u):
            s, h, c0 = decode(u)
            pltpu.async_copy(idx_hbm.at[s, pl.ds(c0, _NG), h],
                             idx_v.at[slot], sem_i[slot])

        def wait_idx(slot):
            pltpu.make_async_copy(idx_hbm.at[0, pl.ds(0, _NG), 0],
                                  idx_v.at[slot], sem_i[slot]).wait()

        def do_adds(slot, u):
            _, h, _ = decode(u)
            off_b = off_v[h, :]
            for j in range(_NG):
                for k in range(_GROW // _LANES):
                    sl = pl.ds(k * _LANES, _LANES)
                    idx_v[slot, j, sl] = idx_v[slot, j, sl] + off_b

        def fire_gathers(slot):
            for j in range(_NG):
                pltpu.async_copy(
                    table_hbm.at[idx_v.at[slot, j]],
                    rows_v.at[slot, pl.ds(j * _GROW, _GROW)],
                    sem_g[slot],
                )

        def wait_gathers(slot):
            pltpu.make_async_copy(table_hbm.at[pl.ds(0, _BQ)],
                                  rows_v.at[slot], sem_g[slot]).wait()

        def do_transpose(slot):
            rows = rows_v.at[slot]

            def tbody(t, _):
                b0 = t * _LANES
                row_idx = b0 + iota16
                c_p = t >> 3
                l0 = (t & 7) * _LANES
                for d in range(_EMBED_DIM):
                    col = jnp.full((_LANES,), d, jnp.int32)
                    v = plsc.load_gather(rows, [row_idx, col])
                    trans_v[slot, d // 8, c_p, d % 8, pl.ds(l0, _LANES)] = v
                return _

            lax.fori_loop(0, _BQ // _LANES, tbody, 0)

        def fire_out(slot, u):
            s, h, c0 = decode(u)
            for r2 in range(2):
                pltpu.async_copy(
                    trans_v.at[slot, r2],
                    out_hbm.at[s, h, r2, pl.ds(c0, _NG)],
                    sem_o[slot],
                )

        def wait_out(slot):
            for r2 in range(2):
                pltpu.make_async_copy(trans_v.at[slot, r2],
                                      out_hbm.at[0, 0, 0, pl.ds(0, _NG)],
                                      sem_o[slot]).wait()

        for b in range(_NBUF):
            fire_idx(b, base_u + b)
        for b in range(_NBUF):
            wait_idx(b)
            do_adds(b, base_u + b)
            fire_gathers(b)
        for b in range(_NBUF):
            wait_gathers(b)
            do_transpose(b)
            fire_out(b, base_u + b)
            fire_idx(b, jnp.minimum(base_u + _NBUF + b, n_units - 1))

        def loop_body(g, _):
            u0 = base_u + _NBUF * g
            for b in range(_NBUF):
                wait_idx(b)
                do_adds(b, u0 + b)
                fire_gathers(b)
            for b in range(_NBUF):
                wait_gathers(b)
                wait_out(b)
                do_transpose(b)
                fire_out(b, u0 + b)
                fire_idx(b, jnp.minimum(u0 + _NBUF + b, n_units - 1))
            return _

        lax.fori_loop(1, units_per_w // _NBUF, loop_body, 0)

        for b in range(_NBUF):
            wait_idx(b)
            wait_out(b)

    return body(table, idx6, off16)


def kernel(table, hash_indices):
    bb, s_dim, h_dim = hash_indices.shape
    idx6 = (hash_indices.astype(jnp.int32)
            .transpose(1, 2, 0)
            .reshape(s_dim, h_dim, bb // 128, 128)
            .transpose(0, 2, 1, 3))
    off16 = jnp.asarray(
        np.repeat(_offsets_np()[:, None], _LANES, axis=1), dtype=jnp.int32)
    n_units = s_dim * h_dim * (bb // _BQ)
    out6 = _sc_gather(table, idx6, off16, n_units // _NW)
    return (out6.transpose(3, 5, 0, 1, 2, 4)
            .reshape(bb, s_dim, h_dim, _EMBED_DIM))
